# Initial kernel scaffold; baseline (speedup 1.0000x reference)
import jax
import jax.numpy as jnp
from jax.experimental import pallas as pl


def kernel(input, table, A, B_w, C_w):
    raise NotImplementedError("write your pallas kernel here")


# speedup vs baseline: 7.3768x; 7.3768x over previous
import functools
import jax
import jax.numpy as jnp
from jax import lax
from jax.experimental import pallas as pl
from jax.experimental.pallas import tpu as pltpu
from jax.experimental.pallas import tpu_sc as plsc

EMBED_DIM = 64
RANK = 16
LANES = 16
NUM_CORES = 2
NUM_SUBCORES = 16
NUM_WORKERS = NUM_CORES * NUM_SUBCORES
CHUNK = 128
GROUP = 8
DBLK = EMBED_DIM // LANES


def _lora_embed(idx3, table, A, M, num_chunks):
    tokens_per_worker = num_chunks * CHUNK
    total = NUM_WORKERS * tokens_per_worker
    mesh = plsc.VectorSubcoreMesh(core_axis_name="c", subcore_axis_name="s")

    @functools.partial(
        pl.kernel,
        mesh=mesh,
        compiler_params=pltpu.CompilerParams(use_tc_tiling_on_sc=False),
        out_type=jax.ShapeDtypeStruct((total, EMBED_DIM), jnp.float32),
        scratch_types=[
            pltpu.VMEM((num_chunks, CHUNK), jnp.int32),
            pltpu.VMEM((CHUNK, EMBED_DIM), jnp.float32),
            pltpu.VMEM((CHUNK, RANK), jnp.float32),
            pltpu.VMEM((RANK, EMBED_DIM), jnp.float32),
            pltpu.SemaphoreType.DMA,
            pltpu.SemaphoreType.DMA,
        ],
    )
    def kern(idx_hbm, table_hbm, a_hbm, m_hbm, out_hbm,
             idx_v, rows_v, a_v, m_v, sem_t, sem_a):
        wid = lax.axis_index("s") * NUM_CORES + lax.axis_index("c")
        base = wid * tokens_per_worker
        pltpu.sync_copy(m_hbm, m_v)
        pltpu.sync_copy(idx_hbm.at[wid], idx_v)

        def chunk_body(j, _):
            cp_t = pltpu.async_copy(table_hbm.at[idx_v.at[j]], rows_v, sem_t)
            cp_a = pltpu.async_copy(a_hbm.at[idx_v.at[j]], a_v, sem_a)
            cp_t.wait()
            cp_a.wait()

            def group_body(g, _):
                t0 = g * GROUP
                accs = [
                    [rows_v[t0 + t, pl.ds(k * LANES, LANES)] for k in range(DBLK)]
                    for t in range(GROUP)
                ]
                a_rows = [a_v[t0 + t, :] for t in range(GROUP)]
                for r in range(RANK):
                    m_vecs = [m_v[r, pl.ds(k * LANES, LANES)] for k in range(DBLK)]
                    for t in range(GROUP):
                        s = a_rows[t][r]
                        for k in range(DBLK):
                            accs[t][k] = accs[t][k] + s * m_vecs[k]
                for t in range(GROUP):
                    for k in range(DBLK):
                        rows_v[t0 + t, pl.ds(k * LANES, LANES)] = accs[t][k]
                return 0

            lax.fori_loop(0, CHUNK // GROUP, group_body, 0)
            pltpu.sync_copy(rows_v, out_hbm.at[pl.ds(base + j * CHUNK, CHUNK)])
            return 0

        lax.fori_loop(0, num_chunks, chunk_body, 0)

    return kern(idx3, table, A, M)


def kernel(input, table, A, B_w, C_w):
    B, L = input.shape
    total = B * L
    assert total % (NUM_WORKERS * CHUNK) == 0
    num_chunks = total // (NUM_WORKERS * CHUNK)
    M = B_w.T @ C_w.T
    idx3 = jnp.reshape(input.astype(jnp.int32), (NUM_WORKERS, num_chunks, CHUNK))
    out = _lora_embed(idx3, table, A, M, num_chunks)
    return jnp.reshape(out, (B, L, EMBED_DIM))


# speedup vs baseline: 8.3009x; 1.1253x over previous
import functools
import jax
import jax.numpy as jnp
from jax import lax
from jax.experimental import pallas as pl
from jax.experimental.pallas import tpu as pltpu
from jax.experimental.pallas import tpu_sc as plsc

EMBED_DIM = 64
RANK = 16
LANES = 16
NUM_CORES = 2
NUM_SUBCORES = 16
NUM_WORKERS = NUM_CORES * NUM_SUBCORES
CHUNK = 128
NBUF = 4
GROUP = 8
DBLK = EMBED_DIM // LANES


def _lora_embed(idx3, table, A, M, num_chunks):
    tokens_per_worker = num_chunks * CHUNK
    total = NUM_WORKERS * tokens_per_worker
    mesh = plsc.VectorSubcoreMesh(core_axis_name="c", subcore_axis_name="s")

    @functools.partial(
        pl.kernel,
        mesh=mesh,
        compiler_params=pltpu.CompilerParams(use_tc_tiling_on_sc=False),
        out_type=jax.ShapeDtypeStruct((total, EMBED_DIM), jnp.float32),
        scratch_types=[
            pltpu.VMEM((num_chunks, CHUNK), jnp.int32),
            pltpu.VMEM((NBUF, CHUNK, EMBED_DIM), jnp.float32),
            pltpu.VMEM((NBUF, CHUNK, RANK), jnp.float32),
            pltpu.VMEM((NBUF, CHUNK, EMBED_DIM), jnp.float32),
            pltpu.VMEM((RANK, EMBED_DIM), jnp.float32),
            pltpu.SemaphoreType.DMA((NBUF,)),
            pltpu.SemaphoreType.DMA((NBUF,)),
        ],
    )
    def kern(idx_hbm, table_hbm, a_hbm, m_hbm, out_hbm,
             idx_v, rows_v, a_v, obuf_v, m_v, gsem, osem):
        wid = lax.axis_index("s") * NUM_CORES + lax.axis_index("c")
        base = wid * tokens_per_worker
        pltpu.sync_copy(m_hbm, m_v)
        pltpu.sync_copy(idx_hbm.at[wid], idx_v)

        def fire_gather(j, b):
            pltpu.async_copy(table_hbm.at[idx_v.at[j]], rows_v.at[b], gsem.at[b])
            pltpu.async_copy(a_hbm.at[idx_v.at[j]], a_v.at[b], gsem.at[b])

        def wait_gather(j, b):
            pltpu.make_async_copy(table_hbm.at[idx_v.at[j]], rows_v.at[b],
                                  gsem.at[b]).wait()
            pltpu.make_async_copy(a_hbm.at[idx_v.at[j]], a_v.at[b],
                                  gsem.at[b]).wait()

        def out_copy(j, b):
            return pltpu.make_async_copy(
                obuf_v.at[b], out_hbm.at[pl.ds(base + j * CHUNK, CHUNK)],
                osem.at[b])

        for b in range(NBUF):
            fire_gather(b, b)

        def compute_chunk(b):
            def group_body(g, _):
                t0 = g * GROUP
                accs = [
                    [rows_v[b, t0 + t, pl.ds(k * LANES, LANES)]
                     for k in range(DBLK)]
                    for t in range(GROUP)
                ]
                a_rows = [a_v[b, t0 + t, :] for t in range(GROUP)]
                for r in range(RANK):
                    m_vecs = [m_v[r, pl.ds(k * LANES, LANES)] for k in range(DBLK)]
                    for t in range(GROUP):
                        s = a_rows[t][r]
                        for k in range(DBLK):
                            accs[t][k] = accs[t][k] + s * m_vecs[k]
                for t in range(GROUP):
                    for k in range(DBLK):
                        obuf_v[b, t0 + t, pl.ds(k * LANES, LANES)] = accs[t][k]
                return 0

            lax.fori_loop(0, CHUNK // GROUP, group_body, 0)

        def ring_body(g, _):
            for b in range(NBUF):
                j = g * NBUF + b
                wait_gather(j, b)

                @pl.when(g > 0)
                def _():
                    out_copy(j - NBUF, b).wait()

                compute_chunk(b)
                out_copy(j, b).start()

                @pl.when(j + NBUF < num_chunks)
                def _():
                    fire_gather(j + NBUF, b)

            return 0

        lax.fori_loop(0, num_chunks // NBUF, ring_body, 0)
        for b in range(NBUF):
            out_copy(num_chunks - NBUF + b, b).wait()

    return kern(idx3, table, A, M)


def kernel(input, table, A, B_w, C_w):
    B, L = input.shape
    total = B * L
    assert total % (NUM_WORKERS * CHUNK * NBUF) == 0
    num_chunks = total // (NUM_WORKERS * CHUNK)
    M = B_w.T @ C_w.T
    idx3 = jnp.reshape(input.astype(jnp.int32), (NUM_WORKERS, num_chunks, CHUNK))
    out = _lora_embed(idx3, table, A, M, num_chunks)
    return jnp.reshape(out, (B, L, EMBED_DIM))


# speedup vs baseline: 8.3127x; 1.0014x over previous
import functools
import jax
import jax.numpy as jnp
from jax import lax
from jax.experimental import pallas as pl
from jax.experimental.pallas import tpu as pltpu
from jax.experimental.pallas import tpu_sc as plsc

EMBED_DIM = 64
RANK = 16
LANES = 16
FROW = 128
NUM_CORES = 2
NUM_SUBCORES = 16
NUM_WORKERS = NUM_CORES * NUM_SUBCORES
BRPC = 4
TC_BLK = 2000


def _build_fused_table(table, A, M):
    V = table.shape[0]

    def body(t_ref, a_ref, m_ref, f_ref):
        c = jnp.dot(a_ref[...], m_ref[...], preferred_element_type=jnp.float32)
        f = t_ref[...] + c
        f_ref[...] = jnp.concatenate([f, jnp.zeros_like(f)], axis=1)

    return pl.pallas_call(
        body,
        grid=(V // TC_BLK,),
        in_specs=[
            pl.BlockSpec((TC_BLK, EMBED_DIM), lambda i: (i, 0)),
            pl.BlockSpec((TC_BLK, RANK), lambda i: (i, 0)),
            pl.BlockSpec((RANK, EMBED_DIM), lambda i: (0, 0)),
        ],
        out_specs=pl.BlockSpec((TC_BLK, FROW), lambda i: (i, 0)),
        out_shape=jax.ShapeDtypeStruct((V, FROW), jnp.float32),
    )(table, A, M)


def _sc_lookup(idx, F, batch, hist):
    br_per_worker = batch // NUM_WORKERS
    num_chunks = br_per_worker // BRPC
    tok = BRPC * hist
    mesh = plsc.VectorSubcoreMesh(core_axis_name="c", subcore_axis_name="s")

    @functools.partial(
        pl.kernel,
        mesh=mesh,
        compiler_params=pltpu.CompilerParams(use_tc_tiling_on_sc=True),
        out_type=jax.ShapeDtypeStruct((batch, hist, EMBED_DIM), jnp.float32),
        scratch_types=[
            pltpu.VMEM((2, BRPC, hist), jnp.int32),
            pltpu.VMEM((2, BRPC, hist, FROW), jnp.float32),
            pltpu.VMEM((2, BRPC, hist, EMBED_DIM), jnp.float32),
            pltpu.SemaphoreType.DMA((2,)),
            pltpu.SemaphoreType.DMA((2,)),
            pltpu.SemaphoreType.DMA((2,)),
        ],
    )
    def kern(idx_hbm, f_hbm, out_hbm, slab_v, rows_v, obuf_v, isem, gsem, osem):
        wid = lax.axis_index("s") * NUM_CORES + lax.axis_index("c")
        br0 = wid * br_per_worker

        def slab_copy(c, b):
            return pltpu.make_async_copy(
                idx_hbm.at[pl.ds(br0 + c * BRPC, BRPC)], slab_v.at[b],
                isem.at[b])

        def gather(b, r):
            return pltpu.make_async_copy(
                f_hbm.at[slab_v.at[b, r]], rows_v.at[b, r], gsem.at[b])

        def out_copy(c, b):
            return pltpu.make_async_copy(
                obuf_v.at[b], out_hbm.at[pl.ds(br0 + c * BRPC, BRPC)],
                osem.at[b])

        slab_copy(0, 0).start()

        def chunk_body(c, _):
            b = lax.rem(c, 2)
            slab_copy(c, b).wait()

            for r in range(BRPC):
                gather(b, r).start()

            @pl.when(c + 1 < num_chunks)
            def _():
                slab_copy(c + 1, 1 - b).start()

            for r in range(BRPC):
                gather(b, r).wait()

            @pl.when(c >= 2)
            def _():
                out_copy(c - 2, b).wait()

            for r in range(BRPC):
                def row_body(l, _):
                    for k in range(EMBED_DIM // LANES):
                        obuf_v[b, r, l, pl.ds(k * LANES, LANES)] = (
                            rows_v[b, r, l, pl.ds(k * LANES, LANES)])
                    return 0
                lax.fori_loop(0, hist, row_body, 0)

            out_copy(c, b).start()
            return 0

        lax.fori_loop(0, num_chunks, chunk_body, 0)
        out_copy(num_chunks - 2, 0).wait()
        out_copy(num_chunks - 1, 1).wait()

    return kern(idx, F)


def kernel(input, table, A, B_w, C_w):
    B, L = input.shape
    M = B_w.T @ C_w.T
    F = _build_fused_table(table, A, M)
    return _sc_lookup(input.astype(jnp.int32), F, B, L)
